# baseline (speedup 1.0000x reference)
import functools

import jax
import jax.numpy as jnp
from jax import lax
from jax.experimental import pallas as pl
from jax.experimental.pallas import tpu as pltpu
from jax.experimental.pallas import tpu_sc as plsc

N = 16384
S = 4096
TN = 1024

_INFO = plsc.get_sparse_core_info()
_NC = _INFO.num_cores
_NS = _INFO.num_subcores
_L = _INFO.num_lanes
_NW = _NC * _NS
NH = N // 2
QPW = NH // _NW


def _tc_body(q_ref, k_ref, qn_ref, kn_ref, i1_ref, i2_ref, i3_ref):
    q = jnp.transpose(q_ref[...], (1, 0))
    qk = jnp.dot(q, k_ref[...], preferred_element_type=jnp.float32)
    sel = (-2.0 * qk + qn_ref[...]) + kn_ref[...]

    iota = lax.broadcasted_iota(jnp.int32, (TN, S), 1)
    inf = jnp.float32(jnp.inf)

    outs = [i1_ref, i2_ref, i3_ref]
    for r in range(3):
        idx = jnp.argmin(sel, axis=1)[:, None]
        outs[r][...] = idx
        if r < 2:
            sel = jnp.where(iota == idx, inf, sel)


def _rsqrt(d2):
    d2 = jnp.maximum(d2, jnp.float32(1e-20))
    i = lax.bitcast_convert_type(d2, jnp.int32)
    i = jnp.int32(0x5F3759DF) - lax.shift_right_logical(i, 1)
    y = lax.bitcast_convert_type(i, jnp.float32)
    for _ in range(3):
        y = y * (jnp.float32(1.5) - jnp.float32(0.5) * d2 * y * y)
    return y


def _sc_kernel(qx_h, qy_h, qz_h, kx_h, ky_h, kz_h, fx_h, fy_h, fz_h,
               i1_h, i2_h, i3_h, ox_h, oy_h, oz_h,
               qx_v, qy_v, qz_v, kx_v, ky_v, kz_v, fx_v, fy_v, fz_v,
               i1_v, i2_v, i3_v, ox_v, oy_v, oz_v):
    wid = lax.axis_index("s") * _NC + lax.axis_index("c")
    base = wid * QPW
    sl_in = pl.ds(base, QPW)
    pltpu.sync_copy(qx_h.at[sl_in], qx_v)
    pltpu.sync_copy(qy_h.at[sl_in], qy_v)
    pltpu.sync_copy(qz_h.at[sl_in], qz_v)
    pltpu.sync_copy(i1_h.at[sl_in], i1_v)
    pltpu.sync_copy(i2_h.at[sl_in], i2_v)
    pltpu.sync_copy(i3_h.at[sl_in], i3_v)
    pltpu.sync_copy(kx_h, kx_v)
    pltpu.sync_copy(ky_h, ky_v)
    pltpu.sync_copy(kz_h, kz_v)
    pltpu.sync_copy(fx_h, fx_v)
    pltpu.sync_copy(fy_h, fy_v)
    pltpu.sync_copy(fz_h, fz_v)

    def body(j, carry):
        sl = pl.ds(j * _L, _L)
        qx = qx_v[sl]
        qy = qy_v[sl]
        qz = qz_v[sl]
        ws = []
        fls = []
        for iv in (i1_v, i2_v, i3_v):
            idx = iv[sl]
            gx = plsc.load_gather(kx_v, [idx])
            gy = plsc.load_gather(ky_v, [idx])
            gz = plsc.load_gather(kz_v, [idx])
            dx = qx - gx
            dy = qy - gy
            dz = qz - gz
            d2 = (dx * dx + dy * dy) + dz * dz
            ws.append(_rsqrt(d2))
            fls.append((plsc.load_gather(fx_v, [idx]),
                        plsc.load_gather(fy_v, [idx]),
                        plsc.load_gather(fz_v, [idx])))
        inv = jnp.float32(1.0) / ((ws[0] + ws[1]) + ws[2])
        w1 = ws[0] * inv
        w2 = ws[1] * inv
        w3 = ws[2] * inv
        ox_v[sl] = (w1 * fls[0][0] + w2 * fls[1][0]) + w3 * fls[2][0]
        oy_v[sl] = (w1 * fls[0][1] + w2 * fls[1][1]) + w3 * fls[2][1]
        oz_v[sl] = (w1 * fls[0][2] + w2 * fls[1][2]) + w3 * fls[2][2]
        return carry

    lax.fori_loop(0, QPW // _L, body, 0)

    pltpu.sync_copy(ox_v, ox_h.at[sl_in])
    pltpu.sync_copy(oy_v, oy_h.at[sl_in])
    pltpu.sync_copy(oz_v, oz_h.at[sl_in])


@functools.partial(jax.jit)
def kernel(xyz, sparse_xyz, sparse_flow):
    qt = xyz[0]
    k = sparse_xyz[0]
    qn = jnp.sum(jnp.transpose(qt, (1, 0)) ** 2, axis=-1)[:, None]
    kn = jnp.sum(jnp.transpose(k, (1, 0)) ** 2, axis=-1)[None, :]

    sc = functools.partial(
        pl.kernel,
        mesh=plsc.VectorSubcoreMesh(core_axis_name="c", subcore_axis_name="s"),
        compiler_params=pltpu.CompilerParams(needs_layout_passes=False),
        out_type=[jax.ShapeDtypeStruct((NH,), jnp.float32)] * 3,
        scratch_types=(
            [pltpu.VMEM((QPW,), jnp.float32)] * 3
            + [pltpu.VMEM((S,), jnp.float32)] * 6
            + [pltpu.VMEM((QPW,), jnp.int32)] * 3
            + [pltpu.VMEM((QPW,), jnp.float32)] * 3
        ),
    )(_sc_kernel)

    halves = []
    for h in range(2):
        qh = lax.slice_in_dim(qt, h * NH, (h + 1) * NH, axis=1)
        qnh = lax.slice_in_dim(qn, h * NH, (h + 1) * NH, axis=0)
        i1, i2, i3 = pl.pallas_call(
            _tc_body,
            grid=(NH // TN,),
            in_specs=[
                pl.BlockSpec((3, TN), lambda i: (0, i)),
                pl.BlockSpec((3, S), lambda i: (0, 0)),
                pl.BlockSpec((TN, 1), lambda i: (i, 0)),
                pl.BlockSpec((1, S), lambda i: (0, 0)),
            ],
            out_specs=[
                pl.BlockSpec((TN, 1), lambda i: (i, 0)),
                pl.BlockSpec((TN, 1), lambda i: (i, 0)),
                pl.BlockSpec((TN, 1), lambda i: (i, 0)),
            ],
            out_shape=[jax.ShapeDtypeStruct((NH, 1), jnp.int32)] * 3,
        )(qh, k, qnh, kn)
        halves.append((qh, i1, i2, i3))

    outs = []
    for h, (qh, i1, i2, i3) in enumerate(halves):
        outs.append(sc(
            qh[0], qh[1], qh[2],
            sparse_xyz[0][0], sparse_xyz[0][1], sparse_xyz[0][2],
            sparse_flow[0][0], sparse_flow[0][1], sparse_flow[0][2],
            jnp.reshape(i1, (NH,)), jnp.reshape(i2, (NH,)),
            jnp.reshape(i3, (NH,)),
        ))
    (ox0, oy0, oz0), (ox1, oy1, oz1) = outs
    out = jnp.stack([jnp.concatenate([ox0, ox1]),
                     jnp.concatenate([oy0, oy1]),
                     jnp.concatenate([oz0, oz1])])
    return out[None]


# speedup vs baseline: 1.0197x; 1.0197x over previous
import functools

import jax
import jax.numpy as jnp
from jax import lax
from jax.experimental import pallas as pl
from jax.experimental.pallas import tpu as pltpu
from jax.experimental.pallas import tpu_sc as plsc

N = 16384
S = 4096
TN = 1024

_INFO = plsc.get_sparse_core_info()
_NC = _INFO.num_cores
_NS = _INFO.num_subcores
_L = _INFO.num_lanes
_NW = _NC * _NS
QPW = N // _NW


def _tc_body(q_ref, k_ref, qn_ref, kn_ref, i1_ref, i2_ref, i3_ref):
    q = jnp.transpose(q_ref[...], (1, 0))
    qk = jnp.dot(q, k_ref[...], preferred_element_type=jnp.float32)
    sel = (-2.0 * qk + qn_ref[...]) + kn_ref[...]

    iota = lax.broadcasted_iota(jnp.int32, (TN, S), 1)
    inf = jnp.float32(jnp.inf)

    outs = [i1_ref, i2_ref, i3_ref]
    for r in range(3):
        idx = jnp.argmin(sel, axis=1)[:, None]
        outs[r][...] = idx
        if r < 2:
            sel = jnp.where(iota == idx, inf, sel)


def _rsqrt(d2):
    d2 = jnp.maximum(d2, jnp.float32(1e-20))
    i = lax.bitcast_convert_type(d2, jnp.int32)
    i = jnp.int32(0x5F3759DF) - lax.shift_right_logical(i, 1)
    y = lax.bitcast_convert_type(i, jnp.float32)
    for _ in range(3):
        y = y * (jnp.float32(1.5) - jnp.float32(0.5) * d2 * y * y)
    return y


def _sc_kernel(qx_h, qy_h, qz_h, kx_h, ky_h, kz_h, fx_h, fy_h, fz_h,
               i1_h, i2_h, i3_h, ox_h, oy_h, oz_h,
               qx_v, qy_v, qz_v, kx_v, ky_v, kz_v, fx_v, fy_v, fz_v,
               i1_v, i2_v, i3_v, ox_v, oy_v, oz_v):
    wid = lax.axis_index("s") * _NC + lax.axis_index("c")
    base = wid * QPW
    sl_in = pl.ds(base, QPW)
    pltpu.sync_copy(qx_h.at[sl_in], qx_v)
    pltpu.sync_copy(qy_h.at[sl_in], qy_v)
    pltpu.sync_copy(qz_h.at[sl_in], qz_v)
    pltpu.sync_copy(i1_h.at[sl_in], i1_v)
    pltpu.sync_copy(i2_h.at[sl_in], i2_v)
    pltpu.sync_copy(i3_h.at[sl_in], i3_v)
    pltpu.sync_copy(kx_h, kx_v)
    pltpu.sync_copy(ky_h, ky_v)
    pltpu.sync_copy(kz_h, kz_v)
    pltpu.sync_copy(fx_h, fx_v)
    pltpu.sync_copy(fy_h, fy_v)
    pltpu.sync_copy(fz_h, fz_v)

    def body(j, carry):
        sl = pl.ds(j * _L, _L)
        qx = qx_v[sl]
        qy = qy_v[sl]
        qz = qz_v[sl]
        ws = []
        fls = []
        for iv in (i1_v, i2_v, i3_v):
            idx = iv[sl]
            gx = plsc.load_gather(kx_v, [idx])
            gy = plsc.load_gather(ky_v, [idx])
            gz = plsc.load_gather(kz_v, [idx])
            dx = qx - gx
            dy = qy - gy
            dz = qz - gz
            d2 = (dx * dx + dy * dy) + dz * dz
            ws.append(_rsqrt(d2))
            fls.append((plsc.load_gather(fx_v, [idx]),
                        plsc.load_gather(fy_v, [idx]),
                        plsc.load_gather(fz_v, [idx])))
        inv = jnp.float32(1.0) / ((ws[0] + ws[1]) + ws[2])
        w1 = ws[0] * inv
        w2 = ws[1] * inv
        w3 = ws[2] * inv
        ox_v[sl] = (w1 * fls[0][0] + w2 * fls[1][0]) + w3 * fls[2][0]
        oy_v[sl] = (w1 * fls[0][1] + w2 * fls[1][1]) + w3 * fls[2][1]
        oz_v[sl] = (w1 * fls[0][2] + w2 * fls[1][2]) + w3 * fls[2][2]
        return carry

    lax.fori_loop(0, QPW // _L, body, 0)

    pltpu.sync_copy(ox_v, ox_h.at[sl_in])
    pltpu.sync_copy(oy_v, oy_h.at[sl_in])
    pltpu.sync_copy(oz_v, oz_h.at[sl_in])


@functools.partial(jax.jit)
def kernel(xyz, sparse_xyz, sparse_flow):
    qt = xyz[0]
    k = sparse_xyz[0]
    qn = jnp.sum(jnp.transpose(qt, (1, 0)) ** 2, axis=-1)[:, None]
    kn = jnp.sum(jnp.transpose(k, (1, 0)) ** 2, axis=-1)[None, :]

    i1, i2, i3 = pl.pallas_call(
        _tc_body,
        grid=(N // TN,),
        in_specs=[
            pl.BlockSpec((3, TN), lambda i: (0, i)),
            pl.BlockSpec((3, S), lambda i: (0, 0)),
            pl.BlockSpec((TN, 1), lambda i: (i, 0)),
            pl.BlockSpec((1, S), lambda i: (0, 0)),
        ],
        out_specs=[
            pl.BlockSpec((TN, 1), lambda i: (i, 0)),
            pl.BlockSpec((TN, 1), lambda i: (i, 0)),
            pl.BlockSpec((TN, 1), lambda i: (i, 0)),
        ],
        out_shape=[jax.ShapeDtypeStruct((N, 1), jnp.int32)] * 3,
    )(qt, k, qn, kn)

    sc = functools.partial(
        pl.kernel,
        mesh=plsc.VectorSubcoreMesh(core_axis_name="c", subcore_axis_name="s"),
        compiler_params=pltpu.CompilerParams(needs_layout_passes=False),
        out_type=[jax.ShapeDtypeStruct((N,), jnp.float32)] * 3,
        scratch_types=(
            [pltpu.VMEM((QPW,), jnp.float32)] * 3
            + [pltpu.VMEM((S,), jnp.float32)] * 6
            + [pltpu.VMEM((QPW,), jnp.int32)] * 3
            + [pltpu.VMEM((QPW,), jnp.float32)] * 3
        ),
    )(_sc_kernel)

    ox, oy, oz = sc(
        xyz[0][0], xyz[0][1], xyz[0][2],
        sparse_xyz[0][0], sparse_xyz[0][1], sparse_xyz[0][2],
        sparse_flow[0][0], sparse_flow[0][1], sparse_flow[0][2],
        jnp.reshape(i1, (N,)), jnp.reshape(i2, (N,)), jnp.reshape(i3, (N,)),
    )
    return jnp.stack([ox, oy, oz])[None]
